# baseline (speedup 1.0000x reference)
import functools

import jax
import jax.numpy as jnp
from jax import lax
from jax.experimental import pallas as pl
from jax.experimental.pallas import tpu as pltpu
from jax.experimental.pallas import tpu_sc as plsc

_G = 8
_NG = 2


def kernel(attribute_value, table):
    idx = jnp.squeeze(attribute_value).astype(jnp.int32)
    (B,) = idx.shape
    V, D = table.shape

    info = plsc.get_sparse_core_info()
    NC, NS, L = info.num_cores, info.num_subcores, info.num_lanes
    NW = NC * NS
    b_per_w = B // NW
    n_blk = b_per_w // L
    n_tr = D // 8

    mesh = plsc.VectorSubcoreMesh(core_axis_name="c", subcore_axis_name="s")

    @functools.partial(
        pl.kernel,
        mesh=mesh,
        out_type=jax.ShapeDtypeStruct((D, B), jnp.float32),
        compiler_params=pltpu.CompilerParams(needs_layout_passes=False),
        scratch_types=[
            pltpu.VMEM((b_per_w,), jnp.int32),
            pltpu.VMEM((_NG, _G, n_tr, 8, 128), jnp.float32),
            pltpu.VMEM((D, b_per_w), jnp.float32),
            pltpu.SemaphoreType.DMA((_NG,)),
        ],
    )
    def gather_kernel(table_hbm, idx_hbm, out_hbm, idx_v, bufs, rows_v, fsem):
        wid = lax.axis_index("s") * NC + lax.axis_index("c")
        base = pl.multiple_of(wid * b_per_w, b_per_w)
        pltpu.sync_copy(idx_hbm.at[pl.ds(base, b_per_w)], idx_v)

        view3 = table_hbm.reshape(n_tr, 8, V)
        c0 = lax.iota(jnp.int32, L)
        t_lo = c0 >> 3
        s_ids = c0 & 7

        def fetch(r, g, slot):
            col = pl.multiple_of((r >> 7) * 128, 128)
            pltpu.async_copy(
                view3.at[:, :, pl.ds(col, 128)], bufs.at[g, slot], fsem.at[g]
            )

        def wait_group(g):
            pltpu.make_async_copy(
                view3.at[:, :, pl.ds(0, _G * 128)], bufs.at[g], fsem.at[g]
            ).wait()

        v0 = idx_v[pl.ds(0, L)]
        for lane in range(L):
            fetch(v0[lane], lane // _G, lane % _G)

        def outer(blk, v_cur):
            v_next = idx_v[pl.ds(lax.min((blk + 1) * L, b_per_w - L), L)]
            for g in range(_NG):
                wait_group(g)
                for s in range(_G):
                    lane = g * _G + s
                    i = blk * L + lane
                    lvec = lax.broadcast(v_cur[lane] & 127, (L,))
                    ivec = lax.broadcast(i, (L,))
                    g0 = plsc.load_gather(bufs.at[g, s], [t_lo, s_ids, lvec])
                    g1 = plsc.load_gather(bufs.at[g, s], [t_lo + 2, s_ids, lvec])
                    plsc.store_scatter(rows_v, [c0, ivec], g0)
                    plsc.store_scatter(rows_v, [c0 + L, ivec], g1)

                @pl.when(blk + 1 < n_blk)
                def _():
                    for s in range(_G):
                        lane = g * _G + s
                        fetch(v_next[lane], g, s)

            return v_next

        lax.fori_loop(0, n_blk, outer, v0)
        pltpu.sync_copy(rows_v, out_hbm.at[:, pl.ds(base, b_per_w)])

    outT = gather_kernel(table.T, idx)
    return outT.T


# speedup vs baseline: 1.1002x; 1.1002x over previous
import functools

import jax
import jax.numpy as jnp
from jax import lax
from jax.experimental import pallas as pl
from jax.experimental.pallas import tpu as pltpu
from jax.experimental.pallas import tpu_sc as plsc

_DEPTH = 8


def kernel(attribute_value, table):
    idx = jnp.squeeze(attribute_value).astype(jnp.int32)
    (B,) = idx.shape
    V, D = table.shape

    info = plsc.get_sparse_core_info()
    NC, NS, L = info.num_cores, info.num_subcores, info.num_lanes
    NW = NC * NS
    b_per_w = B // NW
    n_blk = b_per_w // L

    mesh = plsc.VectorSubcoreMesh(core_axis_name="c", subcore_axis_name="s")

    @functools.partial(
        pl.kernel,
        mesh=mesh,
        out_type=jax.ShapeDtypeStruct((D, B), jnp.float32),
        compiler_params=pltpu.CompilerParams(needs_layout_passes=False),
        scratch_types=[
            pltpu.VMEM((b_per_w,), jnp.int32),
            pltpu.VMEM((_DEPTH, D, 128), jnp.float32),
            pltpu.VMEM((D, b_per_w), jnp.float32),
            pltpu.SemaphoreType.DMA((_DEPTH,)),
        ],
    )
    def gather_kernel(table_hbm, idx_hbm, out_hbm, idx_v, bufs, rows_v, fsem):
        wid = lax.axis_index("s") * NC + lax.axis_index("c")
        base = pl.multiple_of(wid * b_per_w, b_per_w)
        pltpu.sync_copy(idx_hbm.at[pl.ds(base, b_per_w)], idx_v)

        c0 = lax.iota(jnp.int32, L)

        def fetch(r, slot):
            col = pl.multiple_of((r >> 7) * 128, 128)
            pltpu.async_copy(
                table_hbm.at[:, pl.ds(col, 128)], bufs.at[slot], fsem.at[slot]
            )

        v0 = idx_v[pl.ds(0, L)]
        for lane in range(_DEPTH):
            fetch(v0[lane], lane)

        def outer(blk, v_cur):
            v_next = idx_v[pl.ds(lax.min((blk + 1) * L, b_per_w - L), L)]
            for lane in range(L):
                i = blk * L + lane
                slot = lane % _DEPTH
                pltpu.make_async_copy(
                    table_hbm.at[:, pl.ds(0, 128)], bufs.at[slot], fsem.at[slot]
                ).wait()
                lvec = lax.broadcast(v_cur[lane] & 127, (L,))
                ivec = lax.broadcast(i, (L,))
                g0 = plsc.load_gather(bufs.at[slot], [c0, lvec])
                g1 = plsc.load_gather(bufs.at[slot], [c0 + L, lvec])
                plsc.store_scatter(rows_v, [c0, ivec], g0)
                plsc.store_scatter(rows_v, [c0 + L, ivec], g1)
                nxt = v_cur[lane + _DEPTH] if lane < _DEPTH else v_next[lane - _DEPTH]

                @pl.when(i + _DEPTH < b_per_w)
                def _():
                    fetch(nxt, slot)

            return v_next

        lax.fori_loop(0, n_blk, outer, v0)
        pltpu.sync_copy(rows_v, out_hbm.at[:, pl.ds(base, b_per_w)])

    outT = gather_kernel(table.T, idx)
    return outT.T
